# baseline (speedup 1.0000x reference)
import functools

import jax
import jax.numpy as jnp
from jax import lax
from jax.experimental import pallas as pl
from jax.experimental.pallas import tpu as pltpu
from jax.experimental.pallas import tpu_sc as plsc

B = 16384
D = 32
NC = 2
NS = 16
NW = NC * NS
COLS_W = 244
LANES_W = COLS_W * 128
PW = 256
NPIECE = LANES_W // PW
NPTOT = NPIECE + 2
XTRA0 = 999424
TAIL0 = 999936
CM = 2048
CP = 2048
IGN = -1
EPS = 1e-5


def _stream_body(uids, mids, t_mfu, t_mfm, t_mlpu, t_mlpm, fu_out, fm_out,
                 idsb, mj, ml, pj, plb, bufa, bufb, dline, jring, sem, sem2):
    wid = lax.axis_index("s") * NC + lax.axis_index("c")
    lane_lo = wid * LANES_W
    lane_hi = lane_lo + LANES_W
    is0 = wid == 0
    xlo = jnp.where(is0, XTRA0, 0)
    xhi = jnp.where(is0, TAIL0, 0)
    i16 = lax.iota(jnp.int32, 16)

    def run_pass(ids_hbm, ta, tb, fout):
        def count_chunk(c, cnt):
            pltpu.sync_copy(ids_hbm.at[pl.ds(c * 1024, 1024)], idsb)
            def grp(g, cnt):
                v = idsb[pl.ds(g * 16, 16)]
                m = ((v >= lane_lo) & (v < lane_hi)) | ((v >= xlo) & (v < xhi))
                return cnt + plsc.all_reduce_population_count(m)[0]
            return lax.fori_loop(0, 64, grp, cnt)
        cnt_total = lax.fori_loop(0, 16, count_chunk, 0)
        nround = (cnt_total + CM - 1) // CM

        def round_body(r, nflush):
            rlo = r * CM
            rhi = rlo + CM

            def scan_chunk(c, carry):
                pltpu.sync_copy(ids_hbm.at[pl.ds(c * 1024, 1024)], idsb)
                def grp(g, carry):
                    cnt_l, gord = carry
                    v = idsb[pl.ds(g * 16, 16)]
                    m = (((v >= lane_lo) & (v < lane_hi))
                         | ((v >= xlo) & (v < xhi)))
                    cs = plsc.cumsum(m.astype(jnp.int32))
                    o = gord + cs - 1
                    w = m & (o >= rlo) & (o < rhi)
                    jvec = c * 1024 + g * 16 + i16
                    plsc.store_compressed(mj.at[pl.ds(cnt_l, 16)], jvec, mask=w)
                    plsc.store_compressed(ml.at[pl.ds(cnt_l, 16)], v, mask=w)
                    return (cnt_l + plsc.all_reduce_population_count(w)[0],
                            gord + plsc.all_reduce_population_count(m)[0])
                return lax.fori_loop(0, 64, grp, carry)
            cnt_l, _ = lax.fori_loop(0, 16, scan_chunk, (0, 0))
            ngrp = (cnt_l + 15) // 16
            npr = (cnt_l + CP - 1) // CP

            def piece_lo(q):
                return jnp.where(q < NPIECE, lane_lo + q * PW,
                                 XTRA0 + (q - NPIECE) * PW)

            def piece_start(q, par):
                p0 = pl.multiple_of(piece_lo(q), 128)
                pltpu.async_copy(ta.at[:, pl.ds(p0, PW)], bufa.at[par],
                                 sem2.at[par])
                pltpu.async_copy(tb.at[:, pl.ds(p0, PW)], bufb.at[par],
                                 sem2.at[par])

            for qq in range(3):
                piece_start(qq, qq)

            def piece_body(q, nflush):
                par = q & 3
                p0 = pl.multiple_of(piece_lo(q), 128)
                p1 = p0 + PW
                pltpu.make_async_copy(ta.at[:, pl.ds(0, PW)], bufa.at[par],
                                      sem2.at[par]).wait()
                pltpu.make_async_copy(tb.at[:, pl.ds(0, PW)], bufb.at[par],
                                      sem2.at[par]).wait()

                @pl.when(q + 3 < NPTOT)
                def _():
                    piece_start(q + 3, (q + 3) & 3)

                ba = bufa.at[par]
                bb = bufb.at[par]

                def pr_body(pr, nflush):
                    wlo = pr * CP
                    whi = wlo + CP

                    def sub(g, carry):
                        phits, pord = carry
                        vj = mj[pl.ds(g * 16, 16)]
                        vv = ml[pl.ds(g * 16, 16)]
                        m2 = (vv >= p0) & (vv < p1)
                        cs2 = plsc.cumsum(m2.astype(jnp.int32))
                        o2 = pord + cs2 - 1
                        w2 = m2 & (o2 >= wlo) & (o2 < whi)
                        plsc.store_compressed(pj.at[pl.ds(phits, 16)], vj,
                                              mask=w2)
                        plsc.store_compressed(plb.at[pl.ds(phits, 16)],
                                              vv - p0, mask=w2)
                        return (phits + plsc.all_reduce_population_count(w2)[0],
                                pord + plsc.all_reduce_population_count(m2)[0])
                    phits, _ = lax.fori_loop(0, ngrp, sub, (0, 0))

                    pj[pl.ds(phits, 16)] = jnp.full((16,), IGN, jnp.int32)
                    plb[pl.ds(phits, 16)] = jnp.zeros((16,), jnp.int32)

                    def ext(g2, nflush):
                        vjp = pj[pl.ds(g2 * 16, 16)]
                        vlp = plb[pl.ds(g2 * 16, 16)]
                        sel = nflush & 1
                        jring[sel, pl.ds(0, 16)] = vjp
                        base = sel * 16
                        for k in range(16):
                            lv = jnp.broadcast_to(vlp[k], (16,))
                            a0 = plsc.load_gather(ba, [i16, lv])
                            a1 = plsc.load_gather(ba, [i16 + 16, lv])
                            b0 = plsc.load_gather(bb, [i16, lv])
                            b1 = plsc.load_gather(bb, [i16 + 16, lv])
                            row = base + k
                            dline[row, pl.ds(0, 16)] = a0
                            dline[row, pl.ds(16, 16)] = a1
                            dline[row, pl.ds(32, 16)] = b0
                            dline[row, pl.ds(48, 16)] = b1

                        @pl.when(nflush >= 2)
                        def _():
                            pltpu.make_async_copy(
                                fout.at[pl.ds(0, 16)],
                                dline.at[pl.ds(base, 16)], sem).wait()

                        pltpu.async_copy(
                            dline.at[pl.ds(base, 16)],
                            fout.at[plsc.Indices(jring.at[sel],
                                                 ignored_value=IGN)],
                            sem)
                        return nflush + 1
                    return lax.fori_loop(0, (phits + 15) // 16, ext, nflush)
                return lax.fori_loop(0, npr, pr_body, nflush)
            return lax.fori_loop(0, NPTOT, piece_body, nflush)
        nflush = lax.fori_loop(0, nround, round_body, 0)

        for k in range(2):
            @pl.when(nflush >= k + 1)
            def _():
                pltpu.make_async_copy(fout.at[pl.ds(0, 16)],
                                      dline.at[pl.ds(k * 16, 16)], sem).wait()

    run_pass(uids, t_mfu, t_mlpu, fu_out)
    run_pass(mids, t_mfm, t_mlpm, fm_out)


_stream = functools.partial(
    pl.kernel,
    out_type=[jax.ShapeDtypeStruct((B, 128), jnp.float32)] * 2,
    mesh=plsc.VectorSubcoreMesh(core_axis_name="c", subcore_axis_name="s"),
    scratch_types=[
        pltpu.VMEM((1024,), jnp.int32),
        pltpu.VMEM((CM + 16,), jnp.int32),
        pltpu.VMEM((CM + 16,), jnp.int32),
        pltpu.VMEM((CP + 32,), jnp.int32),
        pltpu.VMEM((CP + 32,), jnp.int32),
        pltpu.VMEM((4, 32, PW), jnp.float32),
        pltpu.VMEM((4, 32, PW), jnp.float32),
        pltpu.VMEM((32, 128), jnp.float32),
        pltpu.VMEM((2, 16), jnp.int32),
        pltpu.SemaphoreType.DMA,
        pltpu.SemaphoreType.DMA((4,)),
    ],
    compiler_params=pltpu.CompilerParams(needs_layout_passes=False),
)(_stream_body)


def _dense_body(fu, fm, uid, mid, tailu, tailm, w1u, w1m, c1, w2, c2, w3, c3,
                wfm, wfx, bf, out):
    f32 = jnp.float32
    u = uid[...]
    m = mid[...]
    bs = u.shape[0]
    iot = lax.broadcasted_iota(jnp.int32, (bs, 64), 1)
    oh_u = ((u - TAIL0) == iot).astype(f32)
    oh_m = ((m - TAIL0) == iot).astype(f32)
    tr_u = jnp.dot(oh_u, tailu[...], preferred_element_type=f32)
    tr_m = jnp.dot(oh_m, tailm[...], preferred_element_type=f32)
    selu = u >= TAIL0
    selm = m >= TAIL0
    fu_b = fu[...]
    fm_b = fm[...]
    mfu = jnp.where(selu, tr_u[:, 0:D], fu_b[:, 0:D])
    mlpu = jnp.where(selu, tr_u[:, D:2 * D], fu_b[:, D:2 * D])
    mfm = jnp.where(selm, tr_m[:, 0:D], fm_b[:, 0:D])
    mlpm = jnp.where(selm, tr_m[:, D:2 * D], fm_b[:, D:2 * D])

    x1 = jnp.dot(mlpu, w1u[...], preferred_element_type=f32)
    x1 += jnp.dot(mlpm, w1m[...], preferred_element_type=f32)
    x1 = jnp.maximum(x1 + c1[...], 0.0)
    x2 = jnp.maximum(jnp.dot(x1, w2[...], preferred_element_type=f32) + c2[...], 0.0)
    x3 = jnp.maximum(jnp.dot(x2, w3[...], preferred_element_type=f32) + c3[...], 0.0)
    mf = mfu * mfm
    logit = jnp.dot(mf, wfm[...], preferred_element_type=f32)
    logit += jnp.dot(x3, wfx[...], preferred_element_type=f32)
    logit += bf[...]
    out[...] = jax.nn.sigmoid(logit)


def _dense(fu, fm, uid, mid, tailu, tailm, w1u, w1m, c1, w2, c2, w3, c3,
           wfm, wfx, bf):
    bs = 2048
    grid = (B // bs,)
    line_spec = pl.BlockSpec((bs, 128), lambda i: (i, 0))
    id_spec = pl.BlockSpec((bs, 1), lambda i: (i, 0))
    full = lambda shape: pl.BlockSpec(shape, lambda i: tuple(0 for _ in shape))
    return pl.pallas_call(
        _dense_body,
        grid=grid,
        in_specs=[
            line_spec, line_spec, id_spec, id_spec,
            full((64, 64)), full((64, 64)),
            full((D, 64)), full((D, 64)), full((1, 64)),
            full((64, 32)), full((1, 32)),
            full((32, 16)), full((1, 16)),
            full((D, 1)), full((16, 1)), full((1, 1)),
        ],
        out_specs=pl.BlockSpec((bs, 1), lambda i: (i, 0)),
        out_shape=jax.ShapeDtypeStruct((B, 1), jnp.float32),
    )(fu, fm, uid, mid, tailu, tailm, w1u, w1m, c1, w2, c2, w3, c3,
      wfm, wfx, bf)


def kernel(user_ids, movie_ids, mf_user_emb, mf_movie_emb, mlp_user_emb,
           mlp_movie_emb, W1, b1, g1, bt1, W2, b2, g2, bt2, W3, b3, g3, bt3,
           Wf, bf):
    uids = user_ids.astype(jnp.int32)
    mids = movie_ids.astype(jnp.int32)

    fu, fm = _stream(uids, mids, mf_user_emb.T, mf_movie_emb.T,
                     mlp_user_emb.T, mlp_movie_emb.T)

    tailu = jnp.concatenate(
        [mf_user_emb[TAIL0:], mlp_user_emb[TAIL0:]], axis=1)
    tailm = jnp.concatenate(
        [mf_movie_emb[TAIL0:], mlp_movie_emb[TAIL0:]], axis=1)

    inv = 1.0 / jnp.sqrt(1.0 + EPS)
    a1 = g1 * inv
    a2 = g2 * inv
    a3 = g3 * inv
    w1f = (W1 * a1[:, None]).T
    c1 = (b1 * a1 + bt1)[None, :]
    w2f = (W2 * a2[:, None]).T
    c2 = (b2 * a2 + bt2)[None, :]
    w3f = (W3 * a3[:, None]).T
    c3 = (b3 * a3 + bt3)[None, :]
    wfm = Wf[:, :D].T
    wfx = Wf[:, D:].T
    bfr = bf[None, :]

    return _dense(fu, fm, uids[:, None], mids[:, None], tailu, tailm,
                  w1f[:D], w1f[D:], c1, w2f, c2, w3f, c3, wfm, wfx, bfr)


# speedup vs baseline: 1.2014x; 1.2014x over previous
import functools

import jax
import jax.numpy as jnp
from jax import lax
from jax.experimental import pallas as pl
from jax.experimental.pallas import tpu as pltpu
from jax.experimental.pallas import tpu_sc as plsc

B = 16384
D = 32
NC = 2
NS = 16
NW = NC * NS
COLS_W = 244
LANES_W = COLS_W * 128
NPIECE = 61
XTRA0 = 999424
TAIL0 = 999936
CM = 2048
CP = 2048
IGN = -1
EPS = 1e-5


def _stream_body(uids, mids, t_mfu, t_mfm, t_mlpu, t_mlpm, fu_out, fm_out,
                 idsb, mj, ml, pj, plb, bufa, bufb, dline, jring, sem, sem2):
    wid = lax.axis_index("s") * NC + lax.axis_index("c")
    lane_lo = wid * LANES_W
    lane_hi = lane_lo + LANES_W
    is0 = wid == 0
    xlo = jnp.where(is0, XTRA0, 0)
    xhi = jnp.where(is0, TAIL0, 0)
    i16 = lax.iota(jnp.int32, 16)

    def run_pass(ids_hbm, ta, tb, fout):
        def round_body(r, nflush):
            rlo = r * CM
            rhi = rlo + CM

            def scan_chunk(c, carry):
                pltpu.sync_copy(ids_hbm.at[pl.ds(c * 1024, 1024)], idsb)
                def grp(g, carry):
                    cnt_l, gord = carry
                    v = idsb[pl.ds(g * 16, 16)]
                    m = (((v >= lane_lo) & (v < lane_hi))
                         | ((v >= xlo) & (v < xhi)))
                    cs = plsc.cumsum(m.astype(jnp.int32))
                    o = gord + cs - 1
                    w = m & (o >= rlo) & (o < rhi)
                    jvec = c * 1024 + g * 16 + i16
                    plsc.store_compressed(mj.at[pl.ds(cnt_l, 16)], jvec, mask=w)
                    plsc.store_compressed(ml.at[pl.ds(cnt_l, 16)], v, mask=w)
                    return (cnt_l + plsc.all_reduce_population_count(w)[0],
                            gord + plsc.all_reduce_population_count(m)[0])
                return lax.fori_loop(0, 64, grp, carry)
            cnt_l, cnt_total = lax.fori_loop(0, 16, scan_chunk, (0, 0))
            ngrp = (cnt_l + 15) // 16
            npr = (cnt_l + CP - 1) // CP

            def piece_start(q, par):
                p0 = jnp.where(q < NPIECE, lane_lo + q * 512, XTRA0)
                p0 = pl.multiple_of(p0, 128)
                pltpu.async_copy(ta.at[:, pl.ds(p0, 512)], bufa.at[par],
                                 sem2.at[par])
                pltpu.async_copy(tb.at[:, pl.ds(p0, 512)], bufb.at[par],
                                 sem2.at[par])

            piece_start(0, 0)

            def piece_body(q, nflush):
                par = q & 1
                p0 = jnp.where(q < NPIECE, lane_lo + q * 512, XTRA0)
                p0 = pl.multiple_of(p0, 128)
                p1 = p0 + 512
                pltpu.make_async_copy(ta.at[:, pl.ds(0, 512)], bufa.at[par],
                                      sem2.at[par]).wait()
                pltpu.make_async_copy(tb.at[:, pl.ds(0, 512)], bufb.at[par],
                                      sem2.at[par]).wait()

                @pl.when(q < NPIECE)
                def _():
                    piece_start(q + 1, par ^ 1)

                ba = bufa.at[par]
                bb = bufb.at[par]

                def pr_body(pr, nflush):
                    wlo = pr * CP
                    whi = wlo + CP

                    def sub(g, carry):
                        phits, pord = carry
                        vj = mj[pl.ds(g * 16, 16)]
                        vv = ml[pl.ds(g * 16, 16)]
                        m2 = (vv >= p0) & (vv < p1)
                        cs2 = plsc.cumsum(m2.astype(jnp.int32))
                        o2 = pord + cs2 - 1
                        w2 = m2 & (o2 >= wlo) & (o2 < whi)
                        plsc.store_compressed(pj.at[pl.ds(phits, 16)], vj,
                                              mask=w2)
                        plsc.store_compressed(plb.at[pl.ds(phits, 16)],
                                              vv - p0, mask=w2)
                        return (phits + plsc.all_reduce_population_count(w2)[0],
                                pord + plsc.all_reduce_population_count(m2)[0])
                    phits, _ = lax.fori_loop(0, ngrp, sub, (0, 0))

                    pj[pl.ds(phits, 16)] = jnp.full((16,), IGN, jnp.int32)
                    plb[pl.ds(phits, 16)] = jnp.zeros((16,), jnp.int32)

                    def ext(g2, nflush):
                        vjp = pj[pl.ds(g2 * 16, 16)]
                        vlp = plb[pl.ds(g2 * 16, 16)]
                        sel = nflush & 1
                        jring[sel, pl.ds(0, 16)] = vjp
                        base = sel * 16
                        for k in range(16):
                            lv = jnp.broadcast_to(vlp[k], (16,))
                            a0 = plsc.load_gather(ba, [i16, lv])
                            a1 = plsc.load_gather(ba, [i16 + 16, lv])
                            b0 = plsc.load_gather(bb, [i16, lv])
                            b1 = plsc.load_gather(bb, [i16 + 16, lv])
                            row = base + k
                            dline[row, pl.ds(0, 16)] = a0
                            dline[row, pl.ds(16, 16)] = a1
                            dline[row, pl.ds(32, 16)] = b0
                            dline[row, pl.ds(48, 16)] = b1

                        @pl.when(nflush >= 2)
                        def _():
                            pltpu.make_async_copy(
                                fout.at[pl.ds(0, 16)],
                                dline.at[pl.ds(base, 16)], sem).wait()

                        pltpu.async_copy(
                            dline.at[pl.ds(base, 16)],
                            fout.at[plsc.Indices(jring.at[sel],
                                                 ignored_value=IGN)],
                            sem)
                        return nflush + 1
                    return lax.fori_loop(0, (phits + 15) // 16, ext, nflush)
                return lax.fori_loop(0, npr, pr_body, nflush)
            nflush = lax.fori_loop(0, NPIECE + 1, piece_body, nflush)
            return nflush, cnt_total

        nflush, cnt_total = round_body(0, 0)
        nxtra = jnp.maximum((cnt_total + CM - 1) // CM - 1, 0)

        def extra_round(r, nf):
            nf2, _ = round_body(r + 1, nf)
            return nf2
        nflush = lax.fori_loop(0, nxtra, extra_round, nflush)

        for k in range(2):
            @pl.when(nflush >= k + 1)
            def _():
                pltpu.make_async_copy(fout.at[pl.ds(0, 16)],
                                      dline.at[pl.ds(k * 16, 16)], sem).wait()

    run_pass(uids, t_mfu, t_mlpu, fu_out)
    run_pass(mids, t_mfm, t_mlpm, fm_out)


_stream = functools.partial(
    pl.kernel,
    out_type=[jax.ShapeDtypeStruct((B, 128), jnp.float32)] * 2,
    mesh=plsc.VectorSubcoreMesh(core_axis_name="c", subcore_axis_name="s"),
    scratch_types=[
        pltpu.VMEM((1024,), jnp.int32),
        pltpu.VMEM((CM + 16,), jnp.int32),
        pltpu.VMEM((CM + 16,), jnp.int32),
        pltpu.VMEM((CP + 32,), jnp.int32),
        pltpu.VMEM((CP + 32,), jnp.int32),
        pltpu.VMEM((2, 32, 512), jnp.float32),
        pltpu.VMEM((2, 32, 512), jnp.float32),
        pltpu.VMEM((32, 128), jnp.float32),
        pltpu.VMEM((2, 16), jnp.int32),
        pltpu.SemaphoreType.DMA,
        pltpu.SemaphoreType.DMA((2,)),
    ],
    compiler_params=pltpu.CompilerParams(needs_layout_passes=False),
)(_stream_body)


def _dense_body(fu, fm, uid, mid, tailu, tailm, w1u, w1m, c1, w2, c2, w3, c3,
                wfm, wfx, bf, out):
    f32 = jnp.float32
    u = uid[...]
    m = mid[...]
    bs = u.shape[0]
    iot = lax.broadcasted_iota(jnp.int32, (bs, 64), 1)
    oh_u = ((u - TAIL0) == iot).astype(f32)
    oh_m = ((m - TAIL0) == iot).astype(f32)
    tr_u = jnp.dot(oh_u, tailu[...], preferred_element_type=f32)
    tr_m = jnp.dot(oh_m, tailm[...], preferred_element_type=f32)
    selu = u >= TAIL0
    selm = m >= TAIL0
    fu_b = fu[...]
    fm_b = fm[...]
    mfu = jnp.where(selu, tr_u[:, 0:D], fu_b[:, 0:D])
    mlpu = jnp.where(selu, tr_u[:, D:2 * D], fu_b[:, D:2 * D])
    mfm = jnp.where(selm, tr_m[:, 0:D], fm_b[:, 0:D])
    mlpm = jnp.where(selm, tr_m[:, D:2 * D], fm_b[:, D:2 * D])

    x1 = jnp.dot(mlpu, w1u[...], preferred_element_type=f32)
    x1 += jnp.dot(mlpm, w1m[...], preferred_element_type=f32)
    x1 = jnp.maximum(x1 + c1[...], 0.0)
    x2 = jnp.maximum(jnp.dot(x1, w2[...], preferred_element_type=f32) + c2[...], 0.0)
    x3 = jnp.maximum(jnp.dot(x2, w3[...], preferred_element_type=f32) + c3[...], 0.0)
    mf = mfu * mfm
    logit = jnp.dot(mf, wfm[...], preferred_element_type=f32)
    logit += jnp.dot(x3, wfx[...], preferred_element_type=f32)
    logit += bf[...]
    out[...] = jax.nn.sigmoid(logit)


def _dense(fu, fm, uid, mid, tailu, tailm, w1u, w1m, c1, w2, c2, w3, c3,
           wfm, wfx, bf):
    bs = 4096
    grid = (B // bs,)
    line_spec = pl.BlockSpec((bs, 128), lambda i: (i, 0))
    id_spec = pl.BlockSpec((bs, 1), lambda i: (i, 0))
    full = lambda shape: pl.BlockSpec(shape, lambda i: tuple(0 for _ in shape))
    return pl.pallas_call(
        _dense_body,
        grid=grid,
        in_specs=[
            line_spec, line_spec, id_spec, id_spec,
            full((64, 64)), full((64, 64)),
            full((D, 64)), full((D, 64)), full((1, 64)),
            full((64, 32)), full((1, 32)),
            full((32, 16)), full((1, 16)),
            full((D, 1)), full((16, 1)), full((1, 1)),
        ],
        out_specs=pl.BlockSpec((bs, 1), lambda i: (i, 0)),
        out_shape=jax.ShapeDtypeStruct((B, 1), jnp.float32),
    )(fu, fm, uid, mid, tailu, tailm, w1u, w1m, c1, w2, c2, w3, c3,
      wfm, wfx, bf)


def kernel(user_ids, movie_ids, mf_user_emb, mf_movie_emb, mlp_user_emb,
           mlp_movie_emb, W1, b1, g1, bt1, W2, b2, g2, bt2, W3, b3, g3, bt3,
           Wf, bf):
    uids = user_ids.astype(jnp.int32)
    mids = movie_ids.astype(jnp.int32)

    fu, fm = _stream(uids, mids, mf_user_emb.T, mf_movie_emb.T,
                     mlp_user_emb.T, mlp_movie_emb.T)

    tailu = jnp.concatenate(
        [mf_user_emb[TAIL0:], mlp_user_emb[TAIL0:]], axis=1)
    tailm = jnp.concatenate(
        [mf_movie_emb[TAIL0:], mlp_movie_emb[TAIL0:]], axis=1)

    inv = 1.0 / jnp.sqrt(1.0 + EPS)
    a1 = g1 * inv
    a2 = g2 * inv
    a3 = g3 * inv
    w1f = (W1 * a1[:, None]).T
    c1 = (b1 * a1 + bt1)[None, :]
    w2f = (W2 * a2[:, None]).T
    c2 = (b2 * a2 + bt2)[None, :]
    w3f = (W3 * a3[:, None]).T
    c3 = (b3 * a3 + bt3)[None, :]
    wfm = Wf[:, :D].T
    wfx = Wf[:, D:].T
    bfr = bf[None, :]

    return _dense(fu, fm, uids[:, None], mids[:, None], tailu, tailm,
                  w1f[:D], w1f[D:], c1, w2f, c2, w3f, c3, wfm, wfx, bfr)


# speedup vs baseline: 1.2665x; 1.0541x over previous
import functools

import jax
import jax.numpy as jnp
from jax import lax
from jax.experimental import pallas as pl
from jax.experimental.pallas import tpu as pltpu
from jax.experimental.pallas import tpu_sc as plsc

B = 16384
D = 32
NC = 2
NS = 16
NW = NC * NS
COLS_W = 244
LANES_W = COLS_W * 128
NPIECE = 61
XTRA0 = 999424
TAIL0 = 999936
CM = 2048
CP = 2048
IGN = -1
EPS = 1e-5


def _stream_body(uids, mids, t_mfu, t_mfm, t_mlpu, t_mlpm, fu_out, fm_out,
                 idsb, mj, ml, pj, plb, bufa, bufb, dline, jring, sem, sem2):
    wid = lax.axis_index("s") * NC + lax.axis_index("c")
    lane_lo = wid * LANES_W
    lane_hi = lane_lo + LANES_W
    is0 = wid == 0
    xlo = jnp.where(is0, XTRA0, 0)
    xhi = jnp.where(is0, TAIL0, 0)
    i16 = lax.iota(jnp.int32, 16)

    def run_pass(ids_hbm, ta, tb, fout):
        def round_body(r, nflush):
            rlo = r * CM
            rhi = rlo + CM

            def ids_start(c, par):
                pltpu.async_copy(ids_hbm.at[pl.ds(c * 1024, 1024)],
                                 idsb.at[par], sem2.at[par])

            ids_start(0, 0)

            def scan_chunk(c, carry):
                par = c & 1
                pltpu.make_async_copy(ids_hbm.at[pl.ds(0, 1024)],
                                      idsb.at[par], sem2.at[par]).wait()

                @pl.when(c < 15)
                def _():
                    ids_start(c + 1, par ^ 1)

                def grp(g, carry):
                    cnt_l, gord = carry
                    v = idsb[par, pl.ds(g * 16, 16)]
                    m = (((v >= lane_lo) & (v < lane_hi))
                         | ((v >= xlo) & (v < xhi)))
                    cs = plsc.cumsum(m.astype(jnp.int32))
                    o = gord + cs - 1
                    w = m & (o >= rlo) & (o < rhi)
                    jvec = c * 1024 + g * 16 + i16
                    plsc.store_compressed(mj.at[pl.ds(cnt_l, 16)], jvec, mask=w)
                    plsc.store_compressed(ml.at[pl.ds(cnt_l, 16)], v, mask=w)
                    return (cnt_l + plsc.all_reduce_population_count(w)[0],
                            gord + plsc.all_reduce_population_count(m)[0])
                return lax.fori_loop(0, 64, grp, carry)
            cnt_l, cnt_total = lax.fori_loop(0, 16, scan_chunk, (0, 0))
            ngrp = (cnt_l + 15) // 16
            npr = (cnt_l + CP - 1) // CP

            def piece_start(q, par):
                p0 = jnp.where(q < NPIECE, lane_lo + q * 512, XTRA0)
                p0 = pl.multiple_of(p0, 128)
                pltpu.async_copy(ta.at[:, pl.ds(p0, 512)], bufa.at[par],
                                 sem2.at[par])
                pltpu.async_copy(tb.at[:, pl.ds(p0, 512)], bufb.at[par],
                                 sem2.at[par])

            piece_start(0, 0)

            def piece_body(q, nflush):
                par = q & 1
                p0 = jnp.where(q < NPIECE, lane_lo + q * 512, XTRA0)
                p0 = pl.multiple_of(p0, 128)
                p1 = p0 + 512
                pltpu.make_async_copy(ta.at[:, pl.ds(0, 512)], bufa.at[par],
                                      sem2.at[par]).wait()
                pltpu.make_async_copy(tb.at[:, pl.ds(0, 512)], bufb.at[par],
                                      sem2.at[par]).wait()

                @pl.when(q < NPIECE)
                def _():
                    piece_start(q + 1, par ^ 1)

                ba = bufa.at[par]
                bb = bufb.at[par]

                def pr_body(pr, nflush):
                    wlo = pr * CP
                    whi = wlo + CP

                    def sub(g, carry):
                        phits, pord = carry
                        vj = mj[pl.ds(g * 16, 16)]
                        vv = ml[pl.ds(g * 16, 16)]
                        m2 = (vv >= p0) & (vv < p1)
                        cs2 = plsc.cumsum(m2.astype(jnp.int32))
                        o2 = pord + cs2 - 1
                        w2 = m2 & (o2 >= wlo) & (o2 < whi)
                        plsc.store_compressed(pj.at[pl.ds(phits, 16)], vj,
                                              mask=w2)
                        plsc.store_compressed(plb.at[pl.ds(phits, 16)],
                                              vv - p0, mask=w2)
                        return (phits + plsc.all_reduce_population_count(w2)[0],
                                pord + plsc.all_reduce_population_count(m2)[0])
                    phits, _ = lax.fori_loop(0, ngrp, sub, (0, 0))

                    pj[pl.ds(phits, 16)] = jnp.full((16,), IGN, jnp.int32)
                    plb[pl.ds(phits, 16)] = jnp.zeros((16,), jnp.int32)

                    def ext(g2, nflush):
                        vjp = pj[pl.ds(g2 * 16, 16)]
                        vlp = plb[pl.ds(g2 * 16, 16)]
                        sel = nflush & 1
                        jring[sel, pl.ds(0, 16)] = vjp
                        base = sel * 16
                        for k in range(16):
                            lv = jnp.broadcast_to(vlp[k], (16,))
                            a0 = plsc.load_gather(ba, [i16, lv])
                            a1 = plsc.load_gather(ba, [i16 + 16, lv])
                            b0 = plsc.load_gather(bb, [i16, lv])
                            b1 = plsc.load_gather(bb, [i16 + 16, lv])
                            row = base + k
                            dline[row, pl.ds(0, 16)] = a0
                            dline[row, pl.ds(16, 16)] = a1
                            dline[row, pl.ds(32, 16)] = b0
                            dline[row, pl.ds(48, 16)] = b1

                        @pl.when(nflush >= 2)
                        def _():
                            pltpu.make_async_copy(
                                fout.at[pl.ds(0, 16)],
                                dline.at[pl.ds(base, 16)], sem).wait()

                        pltpu.async_copy(
                            dline.at[pl.ds(base, 16)],
                            fout.at[plsc.Indices(jring.at[sel],
                                                 ignored_value=IGN)],
                            sem)
                        return nflush + 1
                    return lax.fori_loop(0, (phits + 15) // 16, ext, nflush)
                return lax.fori_loop(0, npr, pr_body, nflush)
            nflush = lax.fori_loop(0, NPIECE + 1, piece_body, nflush)
            return nflush, cnt_total

        nflush, cnt_total = round_body(0, 0)
        nxtra = jnp.maximum((cnt_total + CM - 1) // CM - 1, 0)

        def extra_round(r, nf):
            nf2, _ = round_body(r + 1, nf)
            return nf2
        nflush = lax.fori_loop(0, nxtra, extra_round, nflush)

        for k in range(2):
            @pl.when(nflush >= k + 1)
            def _():
                pltpu.make_async_copy(fout.at[pl.ds(0, 16)],
                                      dline.at[pl.ds(k * 16, 16)], sem).wait()

    run_pass(uids, t_mfu, t_mlpu, fu_out)
    run_pass(mids, t_mfm, t_mlpm, fm_out)


_stream = functools.partial(
    pl.kernel,
    out_type=[jax.ShapeDtypeStruct((B, 128), jnp.float32)] * 2,
    mesh=plsc.VectorSubcoreMesh(core_axis_name="c", subcore_axis_name="s"),
    scratch_types=[
        pltpu.VMEM((2, 1024), jnp.int32),
        pltpu.VMEM((CM + 16,), jnp.int32),
        pltpu.VMEM((CM + 16,), jnp.int32),
        pltpu.VMEM((CP + 32,), jnp.int32),
        pltpu.VMEM((CP + 32,), jnp.int32),
        pltpu.VMEM((2, 32, 512), jnp.float32),
        pltpu.VMEM((2, 32, 512), jnp.float32),
        pltpu.VMEM((32, 128), jnp.float32),
        pltpu.VMEM((2, 16), jnp.int32),
        pltpu.SemaphoreType.DMA,
        pltpu.SemaphoreType.DMA((2,)),
    ],
    compiler_params=pltpu.CompilerParams(needs_layout_passes=False),
)(_stream_body)


def _dense_body(fu, fm, uid, mid, tailu, tailm, w1u, w1m, c1, w2, c2, w3, c3,
                wfm, wfx, bf, out):
    f32 = jnp.float32
    u = uid[...]
    m = mid[...]
    bs = u.shape[0]
    iot = lax.broadcasted_iota(jnp.int32, (bs, 64), 1)
    oh_u = ((u - TAIL0) == iot).astype(f32)
    oh_m = ((m - TAIL0) == iot).astype(f32)
    tr_u = jnp.dot(oh_u, tailu[...], preferred_element_type=f32)
    tr_m = jnp.dot(oh_m, tailm[...], preferred_element_type=f32)
    selu = u >= TAIL0
    selm = m >= TAIL0
    fu_b = fu[...]
    fm_b = fm[...]
    mfu = jnp.where(selu, tr_u[:, 0:D], fu_b[:, 0:D])
    mlpu = jnp.where(selu, tr_u[:, D:2 * D], fu_b[:, D:2 * D])
    mfm = jnp.where(selm, tr_m[:, 0:D], fm_b[:, 0:D])
    mlpm = jnp.where(selm, tr_m[:, D:2 * D], fm_b[:, D:2 * D])

    x1 = jnp.dot(mlpu, w1u[...], preferred_element_type=f32)
    x1 += jnp.dot(mlpm, w1m[...], preferred_element_type=f32)
    x1 = jnp.maximum(x1 + c1[...], 0.0)
    x2 = jnp.maximum(jnp.dot(x1, w2[...], preferred_element_type=f32) + c2[...], 0.0)
    x3 = jnp.maximum(jnp.dot(x2, w3[...], preferred_element_type=f32) + c3[...], 0.0)
    mf = mfu * mfm
    logit = jnp.dot(mf, wfm[...], preferred_element_type=f32)
    logit += jnp.dot(x3, wfx[...], preferred_element_type=f32)
    logit += bf[...]
    out[...] = jax.nn.sigmoid(logit)


def _dense(fu, fm, uid, mid, tailu, tailm, w1u, w1m, c1, w2, c2, w3, c3,
           wfm, wfx, bf):
    bs = 4096
    grid = (B // bs,)
    line_spec = pl.BlockSpec((bs, 128), lambda i: (i, 0))
    id_spec = pl.BlockSpec((bs, 1), lambda i: (i, 0))
    full = lambda shape: pl.BlockSpec(shape, lambda i: tuple(0 for _ in shape))
    return pl.pallas_call(
        _dense_body,
        grid=grid,
        in_specs=[
            line_spec, line_spec, id_spec, id_spec,
            full((64, 64)), full((64, 64)),
            full((D, 64)), full((D, 64)), full((1, 64)),
            full((64, 32)), full((1, 32)),
            full((32, 16)), full((1, 16)),
            full((D, 1)), full((16, 1)), full((1, 1)),
        ],
        out_specs=pl.BlockSpec((bs, 1), lambda i: (i, 0)),
        out_shape=jax.ShapeDtypeStruct((B, 1), jnp.float32),
    )(fu, fm, uid, mid, tailu, tailm, w1u, w1m, c1, w2, c2, w3, c3,
      wfm, wfx, bf)


def kernel(user_ids, movie_ids, mf_user_emb, mf_movie_emb, mlp_user_emb,
           mlp_movie_emb, W1, b1, g1, bt1, W2, b2, g2, bt2, W3, b3, g3, bt3,
           Wf, bf):
    uids = user_ids.astype(jnp.int32)
    mids = movie_ids.astype(jnp.int32)

    fu, fm = _stream(uids, mids, mf_user_emb.T, mf_movie_emb.T,
                     mlp_user_emb.T, mlp_movie_emb.T)

    tailu = jnp.concatenate(
        [mf_user_emb[TAIL0:], mlp_user_emb[TAIL0:]], axis=1)
    tailm = jnp.concatenate(
        [mf_movie_emb[TAIL0:], mlp_movie_emb[TAIL0:]], axis=1)

    inv = 1.0 / jnp.sqrt(1.0 + EPS)
    a1 = g1 * inv
    a2 = g2 * inv
    a3 = g3 * inv
    w1f = (W1 * a1[:, None]).T
    c1 = (b1 * a1 + bt1)[None, :]
    w2f = (W2 * a2[:, None]).T
    c2 = (b2 * a2 + bt2)[None, :]
    w3f = (W3 * a3[:, None]).T
    c3 = (b3 * a3 + bt3)[None, :]
    wfm = Wf[:, :D].T
    wfx = Wf[:, D:].T
    bfr = bf[None, :]

    return _dense(fu, fm, uids[:, None], mids[:, None], tailu, tailm,
                  w1f[:D], w1f[D:], c1, w2f, c2, w3f, c3, wfm, wfx, bfr)
